# baseline (speedup 1.0000x reference)
import jax
import jax.numpy as jnp
from jax.experimental import pallas as pl
from jax.experimental.pallas import tpu as pltpu

_Q = 2048
_K = 16384
_D = 64
_KT = 2048
_NT = _K // _KT


def _knn_body(q2_ref, qn_ref, k_ref, ks_ref, kn_ref, ksn_ref, dis_ref, idx_ref,
              bd_ref, bi_ref, bds_ref, bis_ref):
    j = pl.program_id(0)
    q2 = q2_ref[...]
    qn = qn_ref[...]

    def tile_min(k_tile, kn_row):
        qk2 = jax.lax.dot_general(
            q2, k_tile, (((1,), (1,)), ((), ())),
            preferred_element_type=jnp.float32)
        d = (qn + kn_row) + qk2
        m = jnp.min(d, axis=1, keepdims=True)
        a = jnp.argmin(d, axis=1).astype(jnp.int32)[:, None] + j * _KT
        return m, a

    m, a = tile_min(k_ref[...], kn_ref[0])
    ms, as_ = tile_min(ks_ref[...], ksn_ref[0])

    @pl.when(j == 0)
    def _init():
        bd_ref[...], bi_ref[...] = m, a
        bds_ref[...], bis_ref[...] = ms, as_

    @pl.when(j > 0)
    def _update():
        upd = m < bd_ref[...]
        bd_ref[...] = jnp.where(upd, m, bd_ref[...])
        bi_ref[...] = jnp.where(upd, a, bi_ref[...])
        upds = ms < bds_ref[...]
        bds_ref[...] = jnp.where(upds, ms, bds_ref[...])
        bis_ref[...] = jnp.where(upds, as_, bis_ref[...])

    @pl.when(j == _NT - 1)
    def _finish():
        bd, bds = bd_ref[...], bds_ref[...]
        mask = bd < bds
        dis_ref[...] = jnp.where(mask, bd, bds)
        idx_ref[...] = jnp.where(mask, bi_ref[...], bis_ref[...])


def kernel(queries, keys, keys_sym):
    qn = jnp.sum(queries * queries, axis=-1, keepdims=True)
    kn = jnp.sum(keys * keys, axis=-1).reshape(_NT, 1, _KT)
    ksn = jnp.sum(keys_sym * keys_sym, axis=-1).reshape(_NT, 1, _KT)
    q2 = -2.0 * queries
    dis, idx = pl.pallas_call(
        _knn_body,
        grid=(_NT,),
        in_specs=[
            pl.BlockSpec((_Q, _D), lambda j: (0, 0)),
            pl.BlockSpec((_Q, 1), lambda j: (0, 0)),
            pl.BlockSpec((_KT, _D), lambda j: (j, 0)),
            pl.BlockSpec((_KT, _D), lambda j: (j, 0)),
            pl.BlockSpec((1, 1, _KT), lambda j: (j, 0, 0)),
            pl.BlockSpec((1, 1, _KT), lambda j: (j, 0, 0)),
        ],
        out_specs=[
            pl.BlockSpec((_Q, 1), lambda j: (0, 0)),
            pl.BlockSpec((_Q, 1), lambda j: (0, 0)),
        ],
        out_shape=[
            jax.ShapeDtypeStruct((_Q, 1), jnp.float32),
            jax.ShapeDtypeStruct((_Q, 1), jnp.int32),
        ],
        scratch_shapes=[
            pltpu.VMEM((_Q, 1), jnp.float32),
            pltpu.VMEM((_Q, 1), jnp.int32),
            pltpu.VMEM((_Q, 1), jnp.float32),
            pltpu.VMEM((_Q, 1), jnp.int32),
        ],
        compiler_params=pltpu.CompilerParams(
            dimension_semantics=("arbitrary",)),
    )(q2, qn, keys, keys_sym, kn, ksn)
    return dis[:, 0], idx[:, 0]


# speedup vs baseline: 1.1141x; 1.1141x over previous
import jax
import jax.numpy as jnp
from jax.experimental import pallas as pl
from jax.experimental.pallas import tpu as pltpu

_Q = 2048
_K = 16384
_D = 64
_KT = 2048
_NT = _K // _KT


def _knn_body(q2_ref, qn_ref, k_ref, ks_ref, kn_ref, ksn_ref, dis_ref, idx_ref,
              bd_ref, bi_ref):
    j = pl.program_id(0)
    q2 = q2_ref[...]
    qn = qn_ref[...]

    qk2 = jax.lax.dot_general(
        q2, k_ref[...], (((1,), (1,)), ((), ())),
        preferred_element_type=jnp.float32)
    qs2 = jax.lax.dot_general(
        q2, ks_ref[...], (((1,), (1,)), ((), ())),
        preferred_element_type=jnp.float32)
    d = (qn + kn_ref[0]) + qk2
    ds = (qn + ksn_ref[0]) + qs2
    c = jnp.minimum(d, ds)
    m = jnp.min(c, axis=1, keepdims=True)
    a = jnp.argmin(c, axis=1).astype(jnp.int32)[:, None] + j * _KT

    @pl.when(j == 0)
    def _init():
        bd_ref[...], bi_ref[...] = m, a

    @pl.when(j > 0)
    def _update():
        upd = m < bd_ref[...]
        bd_ref[...] = jnp.where(upd, m, bd_ref[...])
        bi_ref[...] = jnp.where(upd, a, bi_ref[...])

    @pl.when(j == _NT - 1)
    def _finish():
        dis_ref[...] = bd_ref[...]
        idx_ref[...] = bi_ref[...]


def kernel(queries, keys, keys_sym):
    qn = jnp.sum(queries * queries, axis=-1, keepdims=True)
    kn = jnp.sum(keys * keys, axis=-1).reshape(_NT, 1, _KT)
    ksn = jnp.sum(keys_sym * keys_sym, axis=-1).reshape(_NT, 1, _KT)
    q2 = -2.0 * queries
    dis, idx = pl.pallas_call(
        _knn_body,
        grid=(_NT,),
        in_specs=[
            pl.BlockSpec((_Q, _D), lambda j: (0, 0)),
            pl.BlockSpec((_Q, 1), lambda j: (0, 0)),
            pl.BlockSpec((_KT, _D), lambda j: (j, 0)),
            pl.BlockSpec((_KT, _D), lambda j: (j, 0)),
            pl.BlockSpec((1, 1, _KT), lambda j: (j, 0, 0)),
            pl.BlockSpec((1, 1, _KT), lambda j: (j, 0, 0)),
        ],
        out_specs=[
            pl.BlockSpec((_Q, 1), lambda j: (0, 0)),
            pl.BlockSpec((_Q, 1), lambda j: (0, 0)),
        ],
        out_shape=[
            jax.ShapeDtypeStruct((_Q, 1), jnp.float32),
            jax.ShapeDtypeStruct((_Q, 1), jnp.int32),
        ],
        scratch_shapes=[
            pltpu.VMEM((_Q, 1), jnp.float32),
            pltpu.VMEM((_Q, 1), jnp.int32),
        ],
        compiler_params=pltpu.CompilerParams(
            dimension_semantics=("arbitrary",)),
    )(q2, qn, keys, keys_sym, kn, ksn)
    return dis[:, 0], idx[:, 0]


# speedup vs baseline: 1.1275x; 1.0120x over previous
import jax
import jax.numpy as jnp
from jax.experimental import pallas as pl
from jax.experimental.pallas import tpu as pltpu

_Q = 2048
_K = 16384
_D = 64
_KT = 1024
_NT = _K // _KT


def _knn_body(q2_ref, qn_ref, k_ref, ks_ref, kn_ref, ksn_ref, dis_ref, idx_ref,
              cbufa_ref, cbufb_ref, bd_ref, bi_ref):
    j = pl.program_id(0)
    even = j % 2 == 0

    def reduce_tile(c_ref):
        c = c_ref[...]
        m = jnp.min(c, axis=1, keepdims=True)
        a = jnp.argmin(c, axis=1).astype(jnp.int32)[:, None] + (j - 1) * _KT

        @pl.when(j == 1)
        def _init():
            bd_ref[...], bi_ref[...] = m, a

        @pl.when(j > 1)
        def _update():
            upd = m < bd_ref[...]
            bd_ref[...] = jnp.where(upd, m, bd_ref[...])
            bi_ref[...] = jnp.where(upd, a, bi_ref[...])

    pl.when((j > 0) & ~even)(lambda: reduce_tile(cbufa_ref))
    pl.when((j > 0) & even)(lambda: reduce_tile(cbufb_ref))

    def form_tile(c_ref):
        q2 = q2_ref[...]
        qn = qn_ref[...]
        qk2 = jax.lax.dot_general(
            q2, k_ref[...], (((1,), (1,)), ((), ())),
            preferred_element_type=jnp.float32)
        qs2 = jax.lax.dot_general(
            q2, ks_ref[...], (((1,), (1,)), ((), ())),
            preferred_element_type=jnp.float32)
        c_ref[...] = jnp.minimum((qn + kn_ref[0]) + qk2,
                                 (qn + ksn_ref[0]) + qs2)

    pl.when((j < _NT) & even)(lambda: form_tile(cbufa_ref))
    pl.when((j < _NT) & ~even)(lambda: form_tile(cbufb_ref))

    @pl.when(j == _NT)
    def _finish():
        dis_ref[...] = bd_ref[...]
        idx_ref[...] = bi_ref[...]


def kernel(queries, keys, keys_sym):
    qn = jnp.sum(queries * queries, axis=-1, keepdims=True)
    kn = jnp.sum(keys * keys, axis=-1).reshape(_NT, 1, _KT)
    ksn = jnp.sum(keys_sym * keys_sym, axis=-1).reshape(_NT, 1, _KT)
    q2 = -2.0 * queries
    _clamp = lambda j: jnp.minimum(j, _NT - 1)
    dis, idx = pl.pallas_call(
        _knn_body,
        grid=(_NT + 1,),
        in_specs=[
            pl.BlockSpec((_Q, _D), lambda j: (0, 0)),
            pl.BlockSpec((_Q, 1), lambda j: (0, 0)),
            pl.BlockSpec((_KT, _D), lambda j: (_clamp(j), 0)),
            pl.BlockSpec((_KT, _D), lambda j: (_clamp(j), 0)),
            pl.BlockSpec((1, 1, _KT), lambda j: (_clamp(j), 0, 0)),
            pl.BlockSpec((1, 1, _KT), lambda j: (_clamp(j), 0, 0)),
        ],
        out_specs=[
            pl.BlockSpec((_Q, 1), lambda j: (0, 0)),
            pl.BlockSpec((_Q, 1), lambda j: (0, 0)),
        ],
        out_shape=[
            jax.ShapeDtypeStruct((_Q, 1), jnp.float32),
            jax.ShapeDtypeStruct((_Q, 1), jnp.int32),
        ],
        scratch_shapes=[
            pltpu.VMEM((_Q, _KT), jnp.float32),
            pltpu.VMEM((_Q, _KT), jnp.float32),
            pltpu.VMEM((_Q, 1), jnp.float32),
            pltpu.VMEM((_Q, 1), jnp.int32),
        ],
        compiler_params=pltpu.CompilerParams(
            dimension_semantics=("arbitrary",)),
    )(q2, qn, keys, keys_sym, kn, ksn)
    return dis[:, 0], idx[:, 0]
